# baseline (speedup 1.0000x reference)
import functools

import jax
import jax.numpy as jnp
from jax import lax
from jax.experimental import pallas as pl
from jax.experimental.pallas import tpu as pltpu
from jax.experimental.pallas import tpu_sc as plsc

B, D, H, W, C = 1, 5, 24, 24, 192
NH, NL, NP = 2, 5, 16
DH = C // NH
HW = H * W
LQ = D * HW
LQP = 3072
HID = C * 4
HLP = NH * NL * NP
QB1 = 768
QB2 = 384
TQ = 128
NSLAB = LQP // TQ
NW = 30


def _stage1_body(xT, w3, b3, WvT, bvb, WoxT, rpxb, WoyT, rpyb, WaT, bab,
                 valT, idxo, wo):
    xb = xT[...]
    mu = jnp.mean(xb, axis=0, keepdims=True)
    var = jnp.mean((xb - mu) ** 2, axis=0, keepdims=True)
    qn = (xb - mu) * lax.rsqrt(var + 1e-5) * w3[...] + b3[...]
    qnb = qn.astype(jnp.bfloat16)
    valT[...] = jnp.dot(WvT[...], qnb, preferred_element_type=jnp.float32) + bvb[...]
    locx = rpxb[...] + jnp.dot(WoxT[...], qnb, preferred_element_type=jnp.float32)
    locy = rpyb[...] + jnp.dot(WoyT[...], qnb, preferred_element_type=jnp.float32)
    logits = jnp.dot(WaT[...], qnb, preferred_element_type=jnp.float32) + bab[...]
    aw_parts = []
    for h in range(NH):
        lh = logits[h * NL * NP:(h + 1) * NL * NP, :]
        m = jnp.max(lh, axis=0, keepdims=True)
        e = jnp.exp(lh - m)
        aw_parts.append(e / jnp.sum(e, axis=0, keepdims=True))
    aw = jnp.concatenate(aw_parts, axis=0)
    gx = locx * float(W) - 0.5
    gy = locy * float(H) - 0.5
    x0 = jnp.floor(gx)
    y0 = jnp.floor(gy)
    wx1 = gx - x0
    wx0 = 1.0 - wx1
    wy1 = gy - y0
    wy0 = 1.0 - wy1
    corners = ((x0, y0, wx0, wy0), (x0 + 1.0, y0, wx1, wy0),
               (x0, y0 + 1.0, wx0, wy1), (x0 + 1.0, y0 + 1.0, wx1, wy1))
    for c, (cx, cy, wx, wy) in enumerate(corners):
        ok = (cx >= 0.0) & (cx <= float(W - 1)) & (cy >= 0.0) & (cy <= float(H - 1))
        ix = jnp.clip(cx, 0.0, float(W - 1)).astype(jnp.int32)
        iy = jnp.clip(cy, 0.0, float(H - 1)).astype(jnp.int32)
        pos = iy * W + ix
        wc = jnp.where(ok, aw * wx * wy, 0.0)
        for hl in range(NH * NL):
            r = hl * 4 * NP + c * NP
            idxo[r:r + NP, :] = pos[hl * NP:(hl + 1) * NP, :]
            wo[r:r + NP, :] = wc[hl * NP:(hl + 1) * NP, :]


def _stage1(xT, w3b, b3b, WvT, bvb, WoxT, rpxb, WoyT, rpyb, WaT, bab):
    nblk = LQP // QB1
    res = functools.partial(pl.BlockSpec, index_map=lambda j: (0, 0))
    col = functools.partial(pl.BlockSpec, index_map=lambda j: (0, j))
    return pl.pallas_call(
        _stage1_body,
        grid=(nblk,),
        in_specs=[
            col((C, QB1)), res((C, QB1)), res((C, QB1)),
            res((C, C)), res((C, QB1)),
            res((HLP, C)), col((HLP, QB1)),
            res((HLP, C)), col((HLP, QB1)),
            res((HLP, C)), res((HLP, QB1)),
        ],
        out_specs=[
            col((C, QB1)),
            col((4 * HLP, QB1)),
            col((4 * HLP, QB1)),
        ],
        out_shape=[
            jax.ShapeDtypeStruct((C, LQP), jnp.float32),
            jax.ShapeDtypeStruct((4 * HLP, LQP), jnp.int32),
            jax.ShapeDtypeStruct((4 * HLP, LQP), jnp.float32),
        ],
    )(xT, w3b, b3b, WvT, bvb, WoxT, rpxb, WoyT, rpyb, WaT, bab)


NSPLIT = 2
NSLABH = NSLAB // NSPLIT
NTASK = NSLABH * NH * NL // NW
HHW = HW // 2


def _sc_body(half, idx_hbm, w_hbm, a_hbm, idx_v, w_v, chunk, in_sem, out_sem):
    wid = lax.axis_index("s") * 2 + lax.axis_index("c")

    @pl.when(wid < NW)
    def _():
        def issue_in(k, buf):
            task = wid * NTASK + k
            t = task // (NH * NL)
            i = task - t * (NH * NL)
            tg = t + half * NSLABH
            cps = (
                pltpu.async_copy(
                    idx_hbm.at[pl.ds(i * 4 * NP, 4 * NP), pl.ds(tg * TQ, TQ)],
                    idx_v.at[buf], in_sem.at[buf]),
                pltpu.async_copy(
                    w_hbm.at[pl.ds(i * 4 * NP, 4 * NP), pl.ds(tg * TQ, TQ)],
                    w_v.at[buf], in_sem.at[buf]),
            )
            return cps, t, i

        zv = jnp.zeros((16,), jnp.float32)
        lane = lax.iota(jnp.int32, 16)

        def zero_half(half):
            def zero_body(z, zc):
                for rr in range(4):
                    for g in range(TQ // 16):
                        chunk[half * HHW + z * 4 + rr, pl.ds(g * 16, 16)] = zv
                return zc
            lax.fori_loop(0, HHW // 4, zero_body, 0)

        pend_in = issue_in(0, 0)
        pend_out = None
        for k in range(NTASK):
            buf = k % 2
            (cp_i, cp_w), t, i = pend_in
            if k + 1 < NTASK:
                pend_in = issue_in(k + 1, 1 - buf)
            if pend_out is None:
                zero_half(0)
                zero_half(1)
            else:
                pend_out[0].wait()
                zero_half(0)
                pend_out[1].wait()
                zero_half(1)
            cp_i.wait()
            cp_w.wait()

            def scat_body(r, sc):
                for g in range(TQ // 16):
                    iv = idx_v[buf, r, pl.ds(g * 16, 16)]
                    wv = w_v[buf, r, pl.ds(g * 16, 16)]
                    plsc.addupdate_scatter(chunk, [iv, lane + (g * 16)], wv)
                return sc
            lax.fori_loop(0, 4 * NP, scat_body, 0)
            pend_out = (
                pltpu.async_copy(chunk.at[pl.ds(0, HHW)],
                                 a_hbm.at[i, t, pl.ds(0, HHW)], out_sem),
                pltpu.async_copy(chunk.at[pl.ds(HHW, HHW)],
                                 a_hbm.at[i, t, pl.ds(HHW, HHW)], out_sem),
            )
        pend_out[0].wait()
        pend_out[1].wait()


def _sc_build_a(idx, w, half):
    sck = pl.kernel(
        functools.partial(_sc_body, half),
        out_type=jax.ShapeDtypeStruct((NH * NL, NSLABH, HW, TQ), jnp.float32),
        mesh=plsc.VectorSubcoreMesh(core_axis_name="c", subcore_axis_name="s",
                                    num_cores=2, num_subcores=16),
        compiler_params=pltpu.CompilerParams(needs_layout_passes=False),
        scratch_types=[
            pltpu.VMEM((2, 4 * NP, TQ), jnp.int32),
            pltpu.VMEM((2, 4 * NP, TQ), jnp.float32),
            pltpu.VMEM((HW, TQ), jnp.float32),
            pltpu.SemaphoreType.DMA((2,)),
            pltpu.SemaphoreType.DMA,
        ],
    )
    return sck(idx, w)


def _stage2_body(valT, A, xT, WoT, bob, w4, b4, fc1T, b1b, fc2T, b2b, outT):
    vt = valT[...]
    heads = []
    for h in range(NH):
        cols = []
        for u in range(QB2 // TQ):
            s = jnp.zeros((DH, TQ), jnp.float32)
            for l in range(NL):
                vhl = vt[h * DH:(h + 1) * DH, l * HW:(l + 1) * HW]
                s = s + jnp.dot(vhl, A[h * NL + l, u],
                                preferred_element_type=jnp.float32)
            cols.append(s)
        heads.append(jnp.concatenate(cols, axis=1))
    attnT = jnp.concatenate(heads, axis=0)
    y = xT[...] + jnp.dot(WoT[...], attnT.astype(jnp.bfloat16),
                          preferred_element_type=jnp.float32) + bob[...]
    mu = jnp.mean(y, axis=0, keepdims=True)
    var = jnp.mean((y - mu) ** 2, axis=0, keepdims=True)
    yn = (y - mu) * lax.rsqrt(var + 1e-5) * w4[...] + b4[...]
    h1 = jnp.dot(fc1T[...], yn.astype(jnp.bfloat16),
                 preferred_element_type=jnp.float32) + b1b[...]
    g = 0.5 * h1 * (1.0 + lax.erf(h1 * 0.7071067811865476))
    ff = jnp.dot(fc2T[...], g.astype(jnp.bfloat16),
                 preferred_element_type=jnp.float32) + b2b[...]
    outT[...] = y + ff


def _stage2(valT, a3, xT, WoT, bob, w4b, b4b, fc1T, b1b, fc2T, b2b, half):
    res = functools.partial(pl.BlockSpec, index_map=lambda j: (0, 0))
    nb = LQP // QB2 // NSPLIT
    col = functools.partial(pl.BlockSpec,
                            index_map=lambda j: (0, j + half * nb))
    out_col = functools.partial(pl.BlockSpec, index_map=lambda j: (0, j))
    return pl.pallas_call(
        _stage2_body,
        grid=(nb,),
        in_specs=[
            res((C, LQP)),
            pl.BlockSpec((NH * NL, QB2 // TQ, HW, TQ), lambda j: (0, j, 0, 0)),
            col((C, QB2)),
            res((C, C)), res((C, QB2)),
            res((C, QB2)), res((C, QB2)),
            res((HID, C)), res((HID, QB2)),
            res((C, HID)), res((C, QB2)),
        ],
        out_specs=[out_col((C, QB2))],
        out_shape=[jax.ShapeDtypeStruct((C, LQP // NSPLIT), jnp.float32)],
    )(valT, a3, xT, WoT, bob, w4b, b4b, fc1T, b1b, fc2T, b2b)[0]


def kernel(x, reference_points, spatial_shapes, level_start_index, norm3_w,
           norm3_b, Wv, bv, Woff, boff, Wattn, battn, Wo, bo, norm4_w,
           norm4_b, fc1_w, fc1_b, fc2_w, fc2_b):
    xT = jnp.pad(x.reshape(LQ, C), ((0, LQP - LQ), (0, 0))).T

    w3b = jnp.broadcast_to(norm3_w[:, None], (C, QB1))
    b3b = jnp.broadcast_to(norm3_b[:, None], (C, QB1))
    WvT = Wv.T.astype(jnp.bfloat16)
    bvb = jnp.broadcast_to(bv[:, None], (C, QB1))

    norm = jnp.stack([spatial_shapes[:, 1], spatial_shapes[:, 0]], -1)
    norm = norm.astype(jnp.float32)
    sx = jnp.broadcast_to(norm[None, :, 0, None], (NH, NL, NP)).reshape(HLP)
    sy = jnp.broadcast_to(norm[None, :, 1, None], (NH, NL, NP)).reshape(HLP)
    Woff3 = Woff.reshape(C, HLP, 2)
    WoxT = (Woff3[:, :, 0] / sx[None, :]).T.astype(jnp.bfloat16)
    WoyT = (Woff3[:, :, 1] / sy[None, :]).T.astype(jnp.bfloat16)
    boff3 = boff.reshape(HLP, 2)
    rp = jnp.pad(reference_points.reshape(LQ, NL, 2),
                 ((0, LQP - LQ), (0, 0), (0, 0)))
    rpx = jnp.broadcast_to(rp[:, None, :, None, 0], (LQP, NH, NL, NP))
    rpy = jnp.broadcast_to(rp[:, None, :, None, 1], (LQP, NH, NL, NP))
    rpxb = rpx.reshape(LQP, HLP).T + (boff3[:, 0] / sx)[:, None]
    rpyb = rpy.reshape(LQP, HLP).T + (boff3[:, 1] / sy)[:, None]
    WaT = Wattn.T.astype(jnp.bfloat16)
    bab = jnp.broadcast_to(battn[:, None], (HLP, QB1))

    valT, idxo, wo = _stage1(xT, w3b, b3b, WvT, bvb, WoxT, rpxb, WoyT, rpyb,
                             WaT, bab)

    a3s = [_sc_build_a(idxo, wo, p) for p in range(NSPLIT)]

    WoT = Wo.T.astype(jnp.bfloat16)
    bob = jnp.broadcast_to(bo[:, None], (C, QB2))
    w4b = jnp.broadcast_to(norm4_w[:, None], (C, QB2))
    b4b = jnp.broadcast_to(norm4_b[:, None], (C, QB2))
    fc1T = fc1_w.T.astype(jnp.bfloat16)
    b1b = jnp.broadcast_to(fc1_b[:, None], (HID, QB2))
    fc2T = fc2_w.T.astype(jnp.bfloat16)
    b2b = jnp.broadcast_to(fc2_b[:, None], (C, QB2))

    outs = [_stage2(valT, a3s[p], xT, WoT, bob, w4b, b4b, fc1T, b1b,
                    fc2T, b2b, p) for p in range(NSPLIT)]
    outT = jnp.concatenate(outs, axis=1)
    return outT[:, :LQ].T.reshape(B, D, H, W, C)


# speedup vs baseline: 1.0223x; 1.0223x over previous
import functools

import jax
import jax.numpy as jnp
from jax import lax
from jax.experimental import pallas as pl
from jax.experimental.pallas import tpu as pltpu
from jax.experimental.pallas import tpu_sc as plsc

B, D, H, W, C = 1, 5, 24, 24, 192
NH, NL, NP = 2, 5, 16
DH = C // NH
HW = H * W
LQ = D * HW
LQP = 3072
HID = C * 4
HLP = NH * NL * NP
QB1 = 768
QB2 = 384
TQ = 128
NSLAB = LQP // TQ
NW = 30


def _stage1_body(xT, w3, b3, WvT, bvb, WoxT, rpxb, WoyT, rpyb, WaT, bab,
                 valT, idxo, wo):
    xb = xT[...]
    mu = jnp.mean(xb, axis=0, keepdims=True)
    var = jnp.mean((xb - mu) ** 2, axis=0, keepdims=True)
    qn = (xb - mu) * lax.rsqrt(var + 1e-5) * w3[...] + b3[...]
    valT[...] = jnp.dot(WvT[...], qn, preferred_element_type=jnp.float32) + bvb[...]
    locx = rpxb[...] + jnp.dot(WoxT[...], qn, preferred_element_type=jnp.float32)
    locy = rpyb[...] + jnp.dot(WoyT[...], qn, preferred_element_type=jnp.float32)
    logits = jnp.dot(WaT[...], qn, preferred_element_type=jnp.float32) + bab[...]
    aw_parts = []
    for h in range(NH):
        lh = logits[h * NL * NP:(h + 1) * NL * NP, :]
        m = jnp.max(lh, axis=0, keepdims=True)
        e = jnp.exp(lh - m)
        aw_parts.append(e / jnp.sum(e, axis=0, keepdims=True))
    aw = jnp.concatenate(aw_parts, axis=0)
    gx = locx * float(W) - 0.5
    gy = locy * float(H) - 0.5
    x0 = jnp.floor(gx)
    y0 = jnp.floor(gy)
    wx1 = gx - x0
    wx0 = 1.0 - wx1
    wy1 = gy - y0
    wy0 = 1.0 - wy1
    corners = ((x0, y0, wx0, wy0), (x0 + 1.0, y0, wx1, wy0),
               (x0, y0 + 1.0, wx0, wy1), (x0 + 1.0, y0 + 1.0, wx1, wy1))
    for c, (cx, cy, wx, wy) in enumerate(corners):
        ok = (cx >= 0.0) & (cx <= float(W - 1)) & (cy >= 0.0) & (cy <= float(H - 1))
        ix = jnp.clip(cx, 0.0, float(W - 1)).astype(jnp.int32)
        iy = jnp.clip(cy, 0.0, float(H - 1)).astype(jnp.int32)
        pos = iy * W + ix
        wc = jnp.where(ok, aw * wx * wy, 0.0)
        for hl in range(NH * NL):
            r = hl * 4 * NP + c * NP
            idxo[r:r + NP, :] = pos[hl * NP:(hl + 1) * NP, :]
            wo[r:r + NP, :] = wc[hl * NP:(hl + 1) * NP, :]


def _stage1(xT, w3b, b3b, WvT, bvb, WoxT, rpxb, WoyT, rpyb, WaT, bab):
    nblk = LQP // QB1
    res = functools.partial(pl.BlockSpec, index_map=lambda j: (0, 0))
    col = functools.partial(pl.BlockSpec, index_map=lambda j: (0, j))
    return pl.pallas_call(
        _stage1_body,
        grid=(nblk,),
        in_specs=[
            col((C, QB1)), res((C, QB1)), res((C, QB1)),
            res((C, C)), res((C, QB1)),
            res((HLP, C)), col((HLP, QB1)),
            res((HLP, C)), col((HLP, QB1)),
            res((HLP, C)), res((HLP, QB1)),
        ],
        out_specs=[
            col((C, QB1)),
            col((4 * HLP, QB1)),
            col((4 * HLP, QB1)),
        ],
        out_shape=[
            jax.ShapeDtypeStruct((C, LQP), jnp.float32),
            jax.ShapeDtypeStruct((4 * HLP, LQP), jnp.int32),
            jax.ShapeDtypeStruct((4 * HLP, LQP), jnp.float32),
        ],
    )(xT, w3b, b3b, WvT, bvb, WoxT, rpxb, WoyT, rpyb, WaT, bab)


NSPLIT = 2
NSLABH = NSLAB // NSPLIT
NTASK = NSLABH * NH * NL // NW
HHW = HW // 2


def _sc_body(half, idx_hbm, w_hbm, a_hbm, idx_v, w_v, chunk, in_sem, out_sem):
    wid = lax.axis_index("s") * 2 + lax.axis_index("c")

    @pl.when(wid < NW)
    def _():
        def issue_in(k, buf):
            task = wid * NTASK + k
            t = task // (NH * NL)
            i = task - t * (NH * NL)
            tg = t + half * NSLABH
            cps = (
                pltpu.async_copy(
                    idx_hbm.at[pl.ds(i * 4 * NP, 4 * NP), pl.ds(tg * TQ, TQ)],
                    idx_v.at[buf], in_sem.at[buf]),
                pltpu.async_copy(
                    w_hbm.at[pl.ds(i * 4 * NP, 4 * NP), pl.ds(tg * TQ, TQ)],
                    w_v.at[buf], in_sem.at[buf]),
            )
            return cps, t, i

        zv = jnp.zeros((16,), jnp.float32)
        lane = lax.iota(jnp.int32, 16)

        def zero_half(half):
            def zero_body(r, zc):
                for g in range(TQ // 16):
                    chunk[r, pl.ds(g * 16, 16)] = zv
                return zc
            lax.fori_loop(half * HHW, (half + 1) * HHW, zero_body, 0)

        pend_in = issue_in(0, 0)
        pend_out = None
        for k in range(NTASK):
            buf = k % 2
            (cp_i, cp_w), t, i = pend_in
            if k + 1 < NTASK:
                pend_in = issue_in(k + 1, 1 - buf)
            if pend_out is None:
                zero_half(0)
                zero_half(1)
            else:
                pend_out[0].wait()
                zero_half(0)
                pend_out[1].wait()
                zero_half(1)
            cp_i.wait()
            cp_w.wait()

            def scat_body(r, sc):
                for g in range(TQ // 16):
                    iv = idx_v[buf, r, pl.ds(g * 16, 16)]
                    wv = w_v[buf, r, pl.ds(g * 16, 16)]
                    plsc.addupdate_scatter(chunk, [iv, lane + (g * 16)], wv)
                return sc
            lax.fori_loop(0, 4 * NP, scat_body, 0)
            pend_out = (
                pltpu.async_copy(chunk.at[pl.ds(0, HHW)],
                                 a_hbm.at[i, t, pl.ds(0, HHW)], out_sem),
                pltpu.async_copy(chunk.at[pl.ds(HHW, HHW)],
                                 a_hbm.at[i, t, pl.ds(HHW, HHW)], out_sem),
            )
        pend_out[0].wait()
        pend_out[1].wait()


def _sc_build_a(idx, w, half):
    sck = pl.kernel(
        functools.partial(_sc_body, half),
        out_type=jax.ShapeDtypeStruct((NH * NL, NSLABH, HW, TQ), jnp.float32),
        mesh=plsc.VectorSubcoreMesh(core_axis_name="c", subcore_axis_name="s",
                                    num_cores=2, num_subcores=16),
        compiler_params=pltpu.CompilerParams(needs_layout_passes=False),
        scratch_types=[
            pltpu.VMEM((2, 4 * NP, TQ), jnp.int32),
            pltpu.VMEM((2, 4 * NP, TQ), jnp.float32),
            pltpu.VMEM((HW, TQ), jnp.float32),
            pltpu.SemaphoreType.DMA((2,)),
            pltpu.SemaphoreType.DMA,
        ],
    )
    return sck(idx, w)


def _stage2_body(valT, A, xT, WoT, bob, w4, b4, fc1T, b1b, fc2T, b2b, outT):
    vt = valT[...]
    heads = []
    for h in range(NH):
        cols = []
        for u in range(QB2 // TQ):
            s = jnp.zeros((DH, TQ), jnp.float32)
            for l in range(NL):
                vhl = vt[h * DH:(h + 1) * DH, l * HW:(l + 1) * HW]
                s = s + jnp.dot(vhl, A[h * NL + l, u],
                                preferred_element_type=jnp.float32)
            cols.append(s)
        heads.append(jnp.concatenate(cols, axis=1))
    attnT = jnp.concatenate(heads, axis=0)
    y = xT[...] + jnp.dot(WoT[...], attnT,
                          preferred_element_type=jnp.float32) + bob[...]
    mu = jnp.mean(y, axis=0, keepdims=True)
    var = jnp.mean((y - mu) ** 2, axis=0, keepdims=True)
    yn = (y - mu) * lax.rsqrt(var + 1e-5) * w4[...] + b4[...]
    h1 = jnp.dot(fc1T[...], yn, preferred_element_type=jnp.float32) + b1b[...]
    g = 0.5 * h1 * (1.0 + lax.erf(h1 * 0.7071067811865476))
    ff = jnp.dot(fc2T[...], g, preferred_element_type=jnp.float32) + b2b[...]
    outT[...] = y + ff


def _stage2(valT, a3, xT, WoT, bob, w4b, b4b, fc1T, b1b, fc2T, b2b, half):
    res = functools.partial(pl.BlockSpec, index_map=lambda j: (0, 0))
    nb = LQP // QB2 // NSPLIT
    col = functools.partial(pl.BlockSpec,
                            index_map=lambda j: (0, j + half * nb))
    out_col = functools.partial(pl.BlockSpec, index_map=lambda j: (0, j))
    return pl.pallas_call(
        _stage2_body,
        grid=(nb,),
        in_specs=[
            res((C, LQP)),
            pl.BlockSpec((NH * NL, QB2 // TQ, HW, TQ), lambda j: (0, j, 0, 0)),
            col((C, QB2)),
            res((C, C)), res((C, QB2)),
            res((C, QB2)), res((C, QB2)),
            res((HID, C)), res((HID, QB2)),
            res((C, HID)), res((C, QB2)),
        ],
        out_specs=[out_col((C, QB2))],
        out_shape=[jax.ShapeDtypeStruct((C, LQP // NSPLIT), jnp.float32)],
    )(valT, a3, xT, WoT, bob, w4b, b4b, fc1T, b1b, fc2T, b2b)[0]


def kernel(x, reference_points, spatial_shapes, level_start_index, norm3_w,
           norm3_b, Wv, bv, Woff, boff, Wattn, battn, Wo, bo, norm4_w,
           norm4_b, fc1_w, fc1_b, fc2_w, fc2_b):
    xT = jnp.pad(x.reshape(LQ, C), ((0, LQP - LQ), (0, 0))).T

    w3b = jnp.broadcast_to(norm3_w[:, None], (C, QB1))
    b3b = jnp.broadcast_to(norm3_b[:, None], (C, QB1))
    WvT = Wv.T
    bvb = jnp.broadcast_to(bv[:, None], (C, QB1))

    norm = jnp.stack([spatial_shapes[:, 1], spatial_shapes[:, 0]], -1)
    norm = norm.astype(jnp.float32)
    sx = jnp.broadcast_to(norm[None, :, 0, None], (NH, NL, NP)).reshape(HLP)
    sy = jnp.broadcast_to(norm[None, :, 1, None], (NH, NL, NP)).reshape(HLP)
    Woff3 = Woff.reshape(C, HLP, 2)
    WoxT = (Woff3[:, :, 0] / sx[None, :]).T
    WoyT = (Woff3[:, :, 1] / sy[None, :]).T
    boff3 = boff.reshape(HLP, 2)
    rp = jnp.pad(reference_points.reshape(LQ, NL, 2),
                 ((0, LQP - LQ), (0, 0), (0, 0)))
    rpx = jnp.broadcast_to(rp[:, None, :, None, 0], (LQP, NH, NL, NP))
    rpy = jnp.broadcast_to(rp[:, None, :, None, 1], (LQP, NH, NL, NP))
    rpxb = rpx.reshape(LQP, HLP).T + (boff3[:, 0] / sx)[:, None]
    rpyb = rpy.reshape(LQP, HLP).T + (boff3[:, 1] / sy)[:, None]
    WaT = Wattn.T
    bab = jnp.broadcast_to(battn[:, None], (HLP, QB1))

    valT, idxo, wo = _stage1(xT, w3b, b3b, WvT, bvb, WoxT, rpxb, WoyT, rpyb,
                             WaT, bab)

    a3s = [_sc_build_a(idxo, wo, p) for p in range(NSPLIT)]

    WoT = Wo.T
    bob = jnp.broadcast_to(bo[:, None], (C, QB2))
    w4b = jnp.broadcast_to(norm4_w[:, None], (C, QB2))
    b4b = jnp.broadcast_to(norm4_b[:, None], (C, QB2))
    fc1T = fc1_w.T
    b1b = jnp.broadcast_to(fc1_b[:, None], (HID, QB2))
    fc2T = fc2_w.T
    b2b = jnp.broadcast_to(fc2_b[:, None], (C, QB2))

    outs = [_stage2(valT, a3s[p], xT, WoT, bob, w4b, b4b, fc1T, b1b,
                    fc2T, b2b, p) for p in range(NSPLIT)]
    outT = jnp.concatenate(outs, axis=1)
    return outT[:, :LQ].T.reshape(B, D, H, W, C)
